# baseline (speedup 1.0000x reference)
import functools
import math

import jax
import jax.numpy as jnp
from jax import lax
from jax.experimental import pallas as pl
from jax.experimental.pallas import tpu as pltpu
from jax.experimental.pallas import tpu_sc as plsc

_SIZE = 100000
_PADDING_IDX = 0
_LABEL_SMOOTHING = 0.1
_SV = _LABEL_SMOOTHING / (_SIZE - 2)
_CONF = 1.0 - _LABEL_SMOOTHING
_C0 = (_SIZE - 2) * _SV * math.log(_SV) + _CONF * math.log(_CONF)

_N = 1024
_BLOCK_V = 2000
_V0 = 68000
_NUM_BLOCKS = _V0 // _BLOCK_V
_SC_V = _SIZE - _V0
_VR = _SC_V // 32
_CH = 96
_NCH = _VR // _CH
_CHREM = _VR - _NCH * _CH

_NC = 2
_NS = 16
_NW = _NC * _NS
_ROWS_PER_W = _N // _NW
_VECS_PER_W = _ROWS_PER_W // 16


def _tc_body(x_ref, t_ref, out_ref):
    k = pl.program_id(0)
    x = x_ref[...]
    t = t_ref[...]
    mask = (t != _PADDING_IDX).astype(jnp.float32)

    @pl.when(k == 0)
    def _first():
        csum = jnp.sum(x, axis=0, keepdims=True) - x[0:1, :]
        partial = jnp.sum(csum * mask, axis=(0, 1), keepdims=True)
        cnt = jnp.sum(mask, axis=(0, 1), keepdims=True)
        out_ref[0] = _C0 * cnt - _SV * partial

    @pl.when(k != 0)
    def _rest():
        csum = jnp.sum(x, axis=0, keepdims=True)
        partial = jnp.sum(csum * mask, axis=(0, 1), keepdims=True)
        out_ref[0] = -_SV * partial


def _tc_partials(xt, trow):
    return pl.pallas_call(
        _tc_body,
        grid=(_NUM_BLOCKS,),
        in_specs=[
            pl.BlockSpec((_BLOCK_V, _N), lambda k: (k, 0)),
            pl.BlockSpec((1, _N), lambda k: (0, 0)),
        ],
        out_specs=pl.BlockSpec((1, 1, 1), lambda k: (k, 0, 0)),
        out_shape=jax.ShapeDtypeStruct((_NUM_BLOCKS, 1, 1), jnp.float32),
        compiler_params=pltpu.CompilerParams(
            dimension_semantics=("parallel",),
        ),
    )(xt, trow)


def _sc_gather_body(x_hbm, tgt_hbm, out_hbm, t_v, tile_v, obuf_v, sem):
    wid = lax.axis_index("s") * _NC + lax.axis_index("c")
    base = wid * _ROWS_PER_W
    pltpu.sync_copy(tgt_hbm.at[pl.ds(base, _ROWS_PER_W)], t_v)
    copies = []
    scalars = []
    for c in range(_VECS_PER_W):
        t16 = t_v[pl.ds(c * 16, 16)]
        for l in range(16):
            r = c * 16 + l
            t_r = t16[l]
            trow0 = pl.multiple_of(t_r - jnp.bitwise_and(t_r, 7), 8)
            col0 = pl.multiple_of((base + r) - (base + r) % 128, 128)
            scalars.append(t_r)
            copies.append(pltpu.async_copy(
                x_hbm.at[pl.ds(trow0, 8), pl.ds(col0, 128)],
                tile_v.at[r], sem))
    for cp in copies:
        cp.wait()
    for r in range(_ROWS_PER_W):
        t_r = scalars[r]
        s_dyn = jnp.bitwise_and(t_r, 7)
        a = ((base + r) % 128) - ((base + r) % 16)
        obuf_v[r] = tile_v[r, s_dyn, pl.ds(a, 16)]
    pltpu.sync_copy(obuf_v, out_hbm.at[pl.ds(base, _ROWS_PER_W)])


def _sc_gather(xt, t32):
    tgt = t32.reshape(_N)
    mesh = plsc.VectorSubcoreMesh(core_axis_name="c", subcore_axis_name="s")
    f = functools.partial(
        pl.kernel,
        mesh=mesh,
        out_type=jax.ShapeDtypeStruct((_N, 16), jnp.float32),
        scratch_types=[
            pltpu.VMEM((_ROWS_PER_W,), jnp.int32),
            pltpu.VMEM((_ROWS_PER_W, 8, 128), jnp.float32),
            pltpu.VMEM((_ROWS_PER_W, 16), jnp.float32),
            pltpu.SemaphoreType.DMA,
        ],
    )(_sc_gather_body)
    return f(xt, tgt)


def _sc_stream_body(x_hbm, out_hbm, buf_v, acc_v, sem):
    wid = lax.axis_index("s") * _NC + lax.axis_index("c")
    row_base = pl.multiple_of(_V0 + wid * _VR, 8)
    z16 = jnp.zeros((16,), jnp.float32)
    for g in range(64):
        acc_v[pl.ds(g * 16, 16)] = z16

    def accumulate(ch_rows):
        for p in range(4):
            def row_body(r, accs):
                return tuple(accs[g] + buf_v[r, pl.ds((p * 16 + g) * 16, 16)]
                             for g in range(16))
            accs = lax.fori_loop(0, ch_rows,
                                 row_body, tuple(z16 for _ in range(16)))
            for g in range(16):
                o = (p * 16 + g) * 16
                acc_v[pl.ds(o, 16)] = acc_v[pl.ds(o, 16)] + accs[g]

    def chunk_body(c, carry):
        r0 = pl.multiple_of(row_base + c * _CH, 8)
        pltpu.async_copy(x_hbm.at[pl.ds(r0, _CH)], buf_v, sem).wait()
        accumulate(_CH)
        return carry

    lax.fori_loop(0, _NCH, chunk_body, 0)
    r0 = pl.multiple_of(row_base + _NCH * _CH, 8)
    pltpu.async_copy(x_hbm.at[pl.ds(r0, _CHREM)],
                     buf_v.at[pl.ds(0, _CHREM)], sem).wait()
    accumulate(_CHREM)
    pltpu.sync_copy(acc_v, out_hbm.at[wid])


def _sc_stream(xt):
    mesh = plsc.VectorSubcoreMesh(core_axis_name="c", subcore_axis_name="s")
    f = functools.partial(
        pl.kernel,
        mesh=mesh,
        out_type=jax.ShapeDtypeStruct((_NW, _N), jnp.float32),
        scratch_types=[
            pltpu.VMEM((_CH, _N), jnp.float32),
            pltpu.VMEM((_N,), jnp.float32),
            pltpu.SemaphoreType.DMA,
        ],
    )(_sc_stream_body)
    return f(xt)


def _comb_body(g_ref, scs_ref, t_ref, trow_ref, out_ref):
    t = t_ref[...]
    mask = (t != _PADDING_IDX).astype(jnp.float32)
    g = g_ref[...]
    rows = jax.lax.broadcasted_iota(jnp.int32, g.shape, 0)
    lanes = jax.lax.broadcasted_iota(jnp.int32, g.shape, 1)
    sel = (lanes == rows % 16).astype(jnp.float32)
    o_t = jnp.sum(g * sel, axis=1, keepdims=True)
    corr = (_SV - _CONF) * jnp.sum(o_t * mask, axis=(0, 1), keepdims=True)
    scs = jnp.sum(scs_ref[...], axis=0, keepdims=True)
    maskr = (trow_ref[...] != _PADDING_IDX).astype(jnp.float32)
    sc_part = jnp.sum(scs * maskr, axis=(0, 1), keepdims=True)
    out_ref[...] = corr - _SV * sc_part


def _combine(gathered, scs, t32, trow):
    return pl.pallas_call(
        _comb_body,
        grid=(1,),
        in_specs=[
            pl.BlockSpec((_N, 16), lambda k: (0, 0)),
            pl.BlockSpec((_NW, _N), lambda k: (0, 0)),
            pl.BlockSpec((_N, 1), lambda k: (0, 0)),
            pl.BlockSpec((1, _N), lambda k: (0, 0)),
        ],
        out_specs=pl.BlockSpec((1, 1), lambda k: (0, 0)),
        out_shape=jax.ShapeDtypeStruct((1, 1), jnp.float32),
    )(gathered, scs, t32, trow)


@jax.jit
def kernel(output, target):
    t32 = target.astype(jnp.int32)
    xt = output.T
    trow = t32.reshape(1, _N)
    gathered = _sc_gather(xt, t32)
    scs = _sc_stream(xt)
    tc = _tc_partials(xt, trow)
    comb = _combine(gathered, scs, t32, trow)
    return jnp.sum(tc) + comb[0, 0]


# speedup vs baseline: 1.0214x; 1.0214x over previous
import functools
import math

import jax
import jax.numpy as jnp
from jax import lax
from jax.experimental import pallas as pl
from jax.experimental.pallas import tpu as pltpu
from jax.experimental.pallas import tpu_sc as plsc

_SIZE = 100000
_PADDING_IDX = 0
_LABEL_SMOOTHING = 0.1
_SV = _LABEL_SMOOTHING / (_SIZE - 2)
_CONF = 1.0 - _LABEL_SMOOTHING
_C0 = (_SIZE - 2) * _SV * math.log(_SV) + _CONF * math.log(_CONF)

_N = 1024
_BLOCK_V = 2000
_V0 = 68000
_NUM_BLOCKS = _V0 // _BLOCK_V
_SC_V = _SIZE - _V0
_VR = _SC_V // 32
_CH = 48
_NCH = _VR // _CH
_CHREM = _VR - _NCH * _CH

_NC = 2
_NS = 16
_NW = _NC * _NS
_ROWS_PER_W = _N // _NW
_VECS_PER_W = _ROWS_PER_W // 16


def _tc_body(x_ref, t_ref, out_ref):
    k = pl.program_id(0)
    x = x_ref[...]
    t = t_ref[...]
    mask = (t != _PADDING_IDX).astype(jnp.float32)

    @pl.when(k == 0)
    def _first():
        csum = jnp.sum(x, axis=0, keepdims=True) - x[0:1, :]
        partial = jnp.sum(csum * mask, axis=(0, 1), keepdims=True)
        cnt = jnp.sum(mask, axis=(0, 1), keepdims=True)
        out_ref[0] = _C0 * cnt - _SV * partial

    @pl.when(k != 0)
    def _rest():
        csum = jnp.sum(x, axis=0, keepdims=True)
        partial = jnp.sum(csum * mask, axis=(0, 1), keepdims=True)
        out_ref[0] = -_SV * partial


def _tc_partials(xt, trow):
    return pl.pallas_call(
        _tc_body,
        grid=(_NUM_BLOCKS,),
        in_specs=[
            pl.BlockSpec((_BLOCK_V, _N), lambda k: (k, 0)),
            pl.BlockSpec((1, _N), lambda k: (0, 0)),
        ],
        out_specs=pl.BlockSpec((1, 1, 1), lambda k: (k, 0, 0)),
        out_shape=jax.ShapeDtypeStruct((_NUM_BLOCKS, 1, 1), jnp.float32),
        compiler_params=pltpu.CompilerParams(
            dimension_semantics=("parallel",),
        ),
    )(xt, trow)


def _sc_gather_body(x_hbm, tgt_hbm, out_hbm, t_v, tile_v, obuf_v, sem):
    wid = lax.axis_index("s") * _NC + lax.axis_index("c")
    base = wid * _ROWS_PER_W
    pltpu.sync_copy(tgt_hbm.at[pl.ds(base, _ROWS_PER_W)], t_v)
    copies = []
    scalars = []
    for c in range(_VECS_PER_W):
        t16 = t_v[pl.ds(c * 16, 16)]
        for l in range(16):
            r = c * 16 + l
            t_r = t16[l]
            trow0 = pl.multiple_of(t_r - jnp.bitwise_and(t_r, 7), 8)
            col0 = pl.multiple_of((base + r) - (base + r) % 128, 128)
            scalars.append(t_r)
            copies.append(pltpu.async_copy(
                x_hbm.at[pl.ds(trow0, 8), pl.ds(col0, 128)],
                tile_v.at[r], sem))
    for cp in copies:
        cp.wait()
    for r in range(_ROWS_PER_W):
        t_r = scalars[r]
        s_dyn = jnp.bitwise_and(t_r, 7)
        a = ((base + r) % 128) - ((base + r) % 16)
        obuf_v[r] = tile_v[r, s_dyn, pl.ds(a, 16)]
    pltpu.sync_copy(obuf_v, out_hbm.at[pl.ds(base, _ROWS_PER_W)])


def _sc_gather(xt, t32):
    tgt = t32.reshape(_N)
    mesh = plsc.VectorSubcoreMesh(core_axis_name="c", subcore_axis_name="s")
    f = functools.partial(
        pl.kernel,
        mesh=mesh,
        out_type=jax.ShapeDtypeStruct((_N, 16), jnp.float32),
        scratch_types=[
            pltpu.VMEM((_ROWS_PER_W,), jnp.int32),
            pltpu.VMEM((_ROWS_PER_W, 8, 128), jnp.float32),
            pltpu.VMEM((_ROWS_PER_W, 16), jnp.float32),
            pltpu.SemaphoreType.DMA,
        ],
    )(_sc_gather_body)
    return f(xt, tgt)


def _sc_stream_body(x_hbm, out_hbm, buf0_v, buf1_v, acc_v, sem0, sem1):
    wid = lax.axis_index("s") * _NC + lax.axis_index("c")
    row_base = pl.multiple_of(_V0 + wid * _VR, 8)
    z16 = jnp.zeros((16,), jnp.float32)
    for g in range(64):
        acc_v[pl.ds(g * 16, 16)] = z16

    bufs = [buf0_v, buf1_v]
    sems = [sem0, sem1]
    ntot = _NCH + (1 if _CHREM else 0)

    def start(c):
        rows = _CH if c < _NCH else _CHREM
        r0 = pl.multiple_of(row_base + c * _CH, 8)
        dst = bufs[c % 2] if rows == _CH else bufs[c % 2].at[pl.ds(0, rows)]
        return pltpu.async_copy(x_hbm.at[pl.ds(r0, rows)], dst, sems[c % 2])

    def accumulate(buf_v, ch_rows):
        for p in range(4):
            def row_body(r, accs):
                return tuple(accs[g] + buf_v[r, pl.ds((p * 16 + g) * 16, 16)]
                             for g in range(16))
            accs = lax.fori_loop(0, ch_rows,
                                 row_body, tuple(z16 for _ in range(16)))
            for g in range(16):
                o = (p * 16 + g) * 16
                acc_v[pl.ds(o, 16)] = acc_v[pl.ds(o, 16)] + accs[g]

    cps = {0: start(0)}
    for c in range(ntot):
        if c + 1 < ntot:
            cps[c + 1] = start(c + 1)
        cps[c].wait()
        accumulate(bufs[c % 2], _CH if c < _NCH else _CHREM)
    pltpu.sync_copy(acc_v, out_hbm.at[wid])


def _sc_stream(xt):
    mesh = plsc.VectorSubcoreMesh(core_axis_name="c", subcore_axis_name="s")
    f = functools.partial(
        pl.kernel,
        mesh=mesh,
        out_type=jax.ShapeDtypeStruct((_NW, _N), jnp.float32),
        scratch_types=[
            pltpu.VMEM((_CH, _N), jnp.float32),
            pltpu.VMEM((_CH, _N), jnp.float32),
            pltpu.VMEM((_N,), jnp.float32),
            pltpu.SemaphoreType.DMA,
            pltpu.SemaphoreType.DMA,
        ],
    )(_sc_stream_body)
    return f(xt)


def _comb_body(g_ref, scs_ref, t_ref, trow_ref, out_ref):
    t = t_ref[...]
    mask = (t != _PADDING_IDX).astype(jnp.float32)
    g = g_ref[...]
    rows = jax.lax.broadcasted_iota(jnp.int32, g.shape, 0)
    lanes = jax.lax.broadcasted_iota(jnp.int32, g.shape, 1)
    sel = (lanes == rows % 16).astype(jnp.float32)
    o_t = jnp.sum(g * sel, axis=1, keepdims=True)
    corr = (_SV - _CONF) * jnp.sum(o_t * mask, axis=(0, 1), keepdims=True)
    scs = jnp.sum(scs_ref[...], axis=0, keepdims=True)
    maskr = (trow_ref[...] != _PADDING_IDX).astype(jnp.float32)
    sc_part = jnp.sum(scs * maskr, axis=(0, 1), keepdims=True)
    out_ref[...] = corr - _SV * sc_part


def _combine(gathered, scs, t32, trow):
    return pl.pallas_call(
        _comb_body,
        grid=(1,),
        in_specs=[
            pl.BlockSpec((_N, 16), lambda k: (0, 0)),
            pl.BlockSpec((_NW, _N), lambda k: (0, 0)),
            pl.BlockSpec((_N, 1), lambda k: (0, 0)),
            pl.BlockSpec((1, _N), lambda k: (0, 0)),
        ],
        out_specs=pl.BlockSpec((1, 1), lambda k: (0, 0)),
        out_shape=jax.ShapeDtypeStruct((1, 1), jnp.float32),
    )(gathered, scs, t32, trow)


@jax.jit
def kernel(output, target):
    t32 = target.astype(jnp.int32)
    xt = output.T
    trow = t32.reshape(1, _N)
    scs = _sc_stream(xt)
    gathered = _sc_gather(xt, t32)
    tc = _tc_partials(xt, trow)
    comb = _combine(gathered, scs, t32, trow)
    return jnp.sum(tc) + comb[0, 0]


# speedup vs baseline: 1.0397x; 1.0179x over previous
import functools
import math

import jax
import jax.numpy as jnp
from jax import lax
from jax.experimental import pallas as pl
from jax.experimental.pallas import tpu as pltpu
from jax.experimental.pallas import tpu_sc as plsc

_SIZE = 100000
_PADDING_IDX = 0
_LABEL_SMOOTHING = 0.1
_SV = _LABEL_SMOOTHING / (_SIZE - 2)
_CONF = 1.0 - _LABEL_SMOOTHING
_C0 = (_SIZE - 2) * _SV * math.log(_SV) + _CONF * math.log(_CONF)

_N = 1024
_BLOCK_V = 2000
_V0 = 68000
_NUM_BLOCKS = _V0 // _BLOCK_V
_SC_V = _SIZE - _V0
_VR = _SC_V // 32
_CH = 48
_NCH = _VR // _CH
_CHREM = _VR - _NCH * _CH

_NC = 2
_NS = 16
_NW = _NC * _NS
_ROWS_PER_W = _N // _NW
_VECS_PER_W = _ROWS_PER_W // 16


def _tc_body(x_ref, t_ref, out_ref):
    k = pl.program_id(0)
    x = x_ref[...]
    t = t_ref[...]
    mask = (t != _PADDING_IDX).astype(jnp.float32)

    @pl.when(k == 0)
    def _first():
        csum = jnp.sum(x, axis=0, keepdims=True) - x[0:1, :]
        partial = jnp.sum(csum * mask, axis=(0, 1), keepdims=True)
        cnt = jnp.sum(mask, axis=(0, 1), keepdims=True)
        out_ref[0] = _C0 * cnt - _SV * partial

    @pl.when(k != 0)
    def _rest():
        csum = jnp.sum(x, axis=0, keepdims=True)
        partial = jnp.sum(csum * mask, axis=(0, 1), keepdims=True)
        out_ref[0] = -_SV * partial


def _tc_partials(xt, trow):
    return pl.pallas_call(
        _tc_body,
        grid=(_NUM_BLOCKS,),
        in_specs=[
            pl.BlockSpec((_BLOCK_V, _N), lambda k: (k, 0)),
            pl.BlockSpec((1, _N), lambda k: (0, 0)),
        ],
        out_specs=pl.BlockSpec((1, 1, 1), lambda k: (k, 0, 0)),
        out_shape=jax.ShapeDtypeStruct((_NUM_BLOCKS, 1, 1), jnp.float32),
        compiler_params=pltpu.CompilerParams(
            dimension_semantics=("parallel",),
        ),
    )(xt, trow)


def _sc_gather_body(x_hbm, tgt_hbm, out_hbm, t_v, tile_v, obuf_v, sem):
    wid = lax.axis_index("s") * _NC + lax.axis_index("c")
    base = wid * _ROWS_PER_W
    pltpu.sync_copy(tgt_hbm.at[pl.ds(base, _ROWS_PER_W)], t_v)
    copies = []
    scalars = []
    for c in range(_VECS_PER_W):
        t16 = t_v[pl.ds(c * 16, 16)]
        for l in range(16):
            r = c * 16 + l
            t_r = t16[l]
            trow0 = pl.multiple_of(t_r - jnp.bitwise_and(t_r, 7), 8)
            col0 = pl.multiple_of((base + r) - (base + r) % 128, 128)
            scalars.append(t_r)
            copies.append(pltpu.async_copy(
                x_hbm.at[pl.ds(trow0, 8), pl.ds(col0, 128)],
                tile_v.at[r], sem))
    for cp in copies:
        cp.wait()
    for r in range(_ROWS_PER_W):
        t_r = scalars[r]
        s_dyn = jnp.bitwise_and(t_r, 7)
        a = ((base + r) % 128) - ((base + r) % 16)
        obuf_v[r] = tile_v[r, s_dyn, pl.ds(a, 16)]
    pltpu.sync_copy(obuf_v, out_hbm.at[pl.ds(base, _ROWS_PER_W)])


def _sc_gather(xt, t32):
    tgt = t32.reshape(_N)
    mesh = plsc.VectorSubcoreMesh(core_axis_name="c", subcore_axis_name="s")
    f = functools.partial(
        pl.kernel,
        mesh=mesh,
        out_type=jax.ShapeDtypeStruct((_N, 16), jnp.float32),
        scratch_types=[
            pltpu.VMEM((_ROWS_PER_W,), jnp.int32),
            pltpu.VMEM((_ROWS_PER_W, 8, 128), jnp.float32),
            pltpu.VMEM((_ROWS_PER_W, 16), jnp.float32),
            pltpu.SemaphoreType.DMA,
        ],
    )(_sc_gather_body)
    return f(xt, tgt)


def _sc_stream_body(x_hbm, out_hbm, buf0_v, buf1_v, acc_v, sem0, sem1):
    wid = lax.axis_index("s") * _NC + lax.axis_index("c")
    row_base = pl.multiple_of(_V0 + wid * _VR, 8)
    z16 = jnp.zeros((16,), jnp.float32)
    for g in range(64):
        acc_v[pl.ds(g * 16, 16)] = z16

    bufs = [buf0_v, buf1_v]
    sems = [sem0, sem1]
    ntot = _NCH + (1 if _CHREM else 0)

    def start(c):
        rows = _CH if c < _NCH else _CHREM
        r0 = pl.multiple_of(row_base + c * _CH, 8)
        dst = bufs[c % 2] if rows == _CH else bufs[c % 2].at[pl.ds(0, rows)]
        return pltpu.async_copy(x_hbm.at[pl.ds(r0, rows)], dst, sems[c % 2])

    def accumulate(buf_v, ch_rows):
        for p in range(4):
            def row_body(r, accs):
                return tuple(accs[g] + buf_v[r, pl.ds((p * 16 + g) * 16, 16)]
                             for g in range(16))
            accs = lax.fori_loop(0, ch_rows,
                                 row_body, tuple(z16 for _ in range(16)))
            for g in range(16):
                o = (p * 16 + g) * 16
                acc_v[pl.ds(o, 16)] = acc_v[pl.ds(o, 16)] + accs[g]

    cps = {0: start(0)}
    for c in range(ntot):
        if c + 1 < ntot:
            cps[c + 1] = start(c + 1)
        cps[c].wait()
        accumulate(bufs[c % 2], _CH if c < _NCH else _CHREM)
    pltpu.sync_copy(acc_v, out_hbm.at[wid])


def _sc_stream(xt):
    mesh = plsc.VectorSubcoreMesh(core_axis_name="c", subcore_axis_name="s")
    f = functools.partial(
        pl.kernel,
        mesh=mesh,
        out_type=jax.ShapeDtypeStruct((_NW, _N), jnp.float32),
        scratch_types=[
            pltpu.VMEM((_CH, _N), jnp.float32),
            pltpu.VMEM((_CH, _N), jnp.float32),
            pltpu.VMEM((_N,), jnp.float32),
            pltpu.SemaphoreType.DMA,
            pltpu.SemaphoreType.DMA,
        ],
    )(_sc_stream_body)
    return f(xt)


def _comb_body(g_ref, scs_ref, t_ref, trow_ref, tc_ref, out_ref):
    t = t_ref[...]
    mask = (t != _PADDING_IDX).astype(jnp.float32)
    g = g_ref[...]
    rows = jax.lax.broadcasted_iota(jnp.int32, g.shape, 0)
    lanes = jax.lax.broadcasted_iota(jnp.int32, g.shape, 1)
    sel = (lanes == rows % 16).astype(jnp.float32)
    o_t = jnp.sum(g * sel, axis=1, keepdims=True)
    corr = (_SV - _CONF) * jnp.sum(o_t * mask, axis=(0, 1), keepdims=True)
    scs = jnp.sum(scs_ref[...], axis=0, keepdims=True)
    maskr = (trow_ref[...] != _PADDING_IDX).astype(jnp.float32)
    sc_part = jnp.sum(scs * maskr, axis=(0, 1), keepdims=True)
    tc_total = jnp.sum(tc_ref[...], axis=(0, 1), keepdims=True)
    out_ref[...] = corr - _SV * sc_part + tc_total


def _combine(gathered, scs, t32, trow, tc):
    return pl.pallas_call(
        _comb_body,
        grid=(1,),
        in_specs=[
            pl.BlockSpec((_N, 16), lambda k: (0, 0)),
            pl.BlockSpec((_NW, _N), lambda k: (0, 0)),
            pl.BlockSpec((_N, 1), lambda k: (0, 0)),
            pl.BlockSpec((1, _N), lambda k: (0, 0)),
            pl.BlockSpec((_NUM_BLOCKS, 1), lambda k: (0, 0)),
        ],
        out_specs=pl.BlockSpec((1, 1), lambda k: (0, 0)),
        out_shape=jax.ShapeDtypeStruct((1, 1), jnp.float32),
    )(gathered, scs, t32, trow, tc)


@jax.jit
def kernel(output, target):
    t32 = target.astype(jnp.int32)
    xt = output.T
    trow = t32.reshape(1, _N)
    scs = _sc_stream(xt)
    gathered = _sc_gather(xt, t32)
    tc = _tc_partials(xt, trow)
    comb = _combine(gathered, scs, t32, trow, tc.reshape(_NUM_BLOCKS, 1))
    return comb[0, 0]


# speedup vs baseline: 1.0427x; 1.0029x over previous
import functools
import math

import jax
import jax.numpy as jnp
from jax import lax
from jax.experimental import pallas as pl
from jax.experimental.pallas import tpu as pltpu
from jax.experimental.pallas import tpu_sc as plsc

_SIZE = 100000
_PADDING_IDX = 0
_LABEL_SMOOTHING = 0.1
_SV = _LABEL_SMOOTHING / (_SIZE - 2)
_CONF = 1.0 - _LABEL_SMOOTHING
_C0 = (_SIZE - 2) * _SV * math.log(_SV) + _CONF * math.log(_CONF)

_N = 1024
_BLOCK_V = 2000
_V0 = 68000
_NUM_BLOCKS = _V0 // _BLOCK_V
_SC_V = _SIZE - _V0
_VR = _SC_V // 32
_CH = 40
_NCH = _VR // _CH

_NC = 2
_NS = 16
_NW = _NC * _NS
_ROWS_PER_W = _N // _NW
_VECS_PER_W = _ROWS_PER_W // 16


def _tc_body(x_ref, t_ref, out_ref):
    k = pl.program_id(0)
    x = x_ref[...]
    t = t_ref[...]
    mask = (t != _PADDING_IDX).astype(jnp.float32)

    @pl.when(k == 0)
    def _first():
        csum = jnp.sum(x, axis=0, keepdims=True) - x[0:1, :]
        partial = jnp.sum(csum * mask, axis=(0, 1), keepdims=True)
        cnt = jnp.sum(mask, axis=(0, 1), keepdims=True)
        out_ref[0] = _C0 * cnt - _SV * partial

    @pl.when(k != 0)
    def _rest():
        csum = jnp.sum(x, axis=0, keepdims=True)
        partial = jnp.sum(csum * mask, axis=(0, 1), keepdims=True)
        out_ref[0] = -_SV * partial


def _tc_partials(xt, trow):
    return pl.pallas_call(
        _tc_body,
        grid=(_NUM_BLOCKS,),
        in_specs=[
            pl.BlockSpec((_BLOCK_V, _N), lambda k: (k, 0)),
            pl.BlockSpec((1, _N), lambda k: (0, 0)),
        ],
        out_specs=pl.BlockSpec((1, 1, 1), lambda k: (k, 0, 0)),
        out_shape=jax.ShapeDtypeStruct((_NUM_BLOCKS, 1, 1), jnp.float32),
        compiler_params=pltpu.CompilerParams(
            dimension_semantics=("parallel",),
        ),
    )(xt, trow)


def _sc_body(x_hbm, tgt_hbm, gout_hbm, sout_hbm,
             t_v, tile_v, obuf_v, buf0_v, buf1_v, acc_v,
             semg, sem0, sem1):
    wid = lax.axis_index("s") * _NC + lax.axis_index("c")
    base = wid * _ROWS_PER_W
    pltpu.sync_copy(tgt_hbm.at[pl.ds(base, _ROWS_PER_W)], t_v)

    gcopies = []
    scalars = []
    for c in range(_VECS_PER_W):
        t16 = t_v[pl.ds(c * 16, 16)]
        for l in range(16):
            r = c * 16 + l
            t_r = t16[l]
            trow0 = pl.multiple_of(t_r - jnp.bitwise_and(t_r, 7), 8)
            col0 = pl.multiple_of((base + r) - (base + r) % 128, 128)
            scalars.append(t_r)
            gcopies.append(pltpu.async_copy(
                x_hbm.at[pl.ds(trow0, 8), pl.ds(col0, 128)],
                tile_v.at[r], semg))

    row_base = pl.multiple_of(_V0 + wid * _VR, 8)
    z16 = jnp.zeros((16,), jnp.float32)
    for g in range(64):
        acc_v[pl.ds(g * 16, 16)] = z16
    bufs = [buf0_v, buf1_v]
    sems = [sem0, sem1]

    def start(c):
        r0 = pl.multiple_of(row_base + c * _CH, 8)
        return pltpu.async_copy(x_hbm.at[pl.ds(r0, _CH)], bufs[c % 2],
                                sems[c % 2])

    def accumulate(buf_v):
        for p in range(4):
            def row_body(r, accs):
                return tuple(accs[g] + buf_v[r, pl.ds((p * 16 + g) * 16, 16)]
                             for g in range(16))
            accs = lax.fori_loop(0, _CH,
                                 row_body, tuple(z16 for _ in range(16)))
            for g in range(16):
                o = (p * 16 + g) * 16
                acc_v[pl.ds(o, 16)] = acc_v[pl.ds(o, 16)] + accs[g]

    cps = {0: start(0)}
    for c in range(_NCH):
        if c + 1 < _NCH:
            cps[c + 1] = start(c + 1)
        cps[c].wait()
        accumulate(bufs[c % 2])
    pltpu.sync_copy(acc_v, sout_hbm.at[wid])

    for cp in gcopies:
        cp.wait()
    for r in range(_ROWS_PER_W):
        t_r = scalars[r]
        s_dyn = jnp.bitwise_and(t_r, 7)
        a = ((base + r) % 128) - ((base + r) % 16)
        obuf_v[r] = tile_v[r, s_dyn, pl.ds(a, 16)]
    pltpu.sync_copy(obuf_v, gout_hbm.at[pl.ds(base, _ROWS_PER_W)])


def _sc_all(xt, t32):
    tgt = t32.reshape(_N)
    mesh = plsc.VectorSubcoreMesh(core_axis_name="c", subcore_axis_name="s")
    f = functools.partial(
        pl.kernel,
        mesh=mesh,
        out_type=(jax.ShapeDtypeStruct((_N, 16), jnp.float32),
                  jax.ShapeDtypeStruct((_NW, _N), jnp.float32)),
        scratch_types=[
            pltpu.VMEM((_ROWS_PER_W,), jnp.int32),
            pltpu.VMEM((_ROWS_PER_W, 8, 128), jnp.float32),
            pltpu.VMEM((_ROWS_PER_W, 16), jnp.float32),
            pltpu.VMEM((_CH, _N), jnp.float32),
            pltpu.VMEM((_CH, _N), jnp.float32),
            pltpu.VMEM((_N,), jnp.float32),
            pltpu.SemaphoreType.DMA,
            pltpu.SemaphoreType.DMA,
            pltpu.SemaphoreType.DMA,
        ],
    )(_sc_body)
    return f(xt, tgt)


def _comb_body(g_ref, scs_ref, t_ref, trow_ref, tc_ref, out_ref):
    t = t_ref[...]
    mask = (t != _PADDING_IDX).astype(jnp.float32)
    g = g_ref[...]
    rows = jax.lax.broadcasted_iota(jnp.int32, g.shape, 0)
    lanes = jax.lax.broadcasted_iota(jnp.int32, g.shape, 1)
    sel = (lanes == rows % 16).astype(jnp.float32)
    o_t = jnp.sum(g * sel, axis=1, keepdims=True)
    corr = (_SV - _CONF) * jnp.sum(o_t * mask, axis=(0, 1), keepdims=True)
    scs = jnp.sum(scs_ref[...], axis=0, keepdims=True)
    maskr = (trow_ref[...] != _PADDING_IDX).astype(jnp.float32)
    sc_part = jnp.sum(scs * maskr, axis=(0, 1), keepdims=True)
    tc_total = jnp.sum(tc_ref[...], axis=(0, 1), keepdims=True)
    out_ref[...] = corr - _SV * sc_part + tc_total


def _combine(gathered, scs, t32, trow, tc):
    return pl.pallas_call(
        _comb_body,
        grid=(1,),
        in_specs=[
            pl.BlockSpec((_N, 16), lambda k: (0, 0)),
            pl.BlockSpec((_NW, _N), lambda k: (0, 0)),
            pl.BlockSpec((_N, 1), lambda k: (0, 0)),
            pl.BlockSpec((1, _N), lambda k: (0, 0)),
            pl.BlockSpec((_NUM_BLOCKS, 1), lambda k: (0, 0)),
        ],
        out_specs=pl.BlockSpec((1, 1), lambda k: (0, 0)),
        out_shape=jax.ShapeDtypeStruct((1, 1), jnp.float32),
    )(gathered, scs, t32, trow, tc)


@jax.jit
def kernel(output, target):
    t32 = target.astype(jnp.int32)
    xt = output.T
    trow = t32.reshape(1, _N)
    gathered, scs = _sc_all(xt, t32)
    tc = _tc_partials(xt, trow)
    comb = _combine(gathered, scs, t32, trow, tc.reshape(_NUM_BLOCKS, 1))
    return comb[0, 0]


# speedup vs baseline: 1.0474x; 1.0045x over previous
import functools
import math

import jax
import jax.numpy as jnp
from jax import lax
from jax.experimental import pallas as pl
from jax.experimental.pallas import tpu as pltpu
from jax.experimental.pallas import tpu_sc as plsc

_SIZE = 100000
_PADDING_IDX = 0
_LABEL_SMOOTHING = 0.1
_SV = _LABEL_SMOOTHING / (_SIZE - 2)
_CONF = 1.0 - _LABEL_SMOOTHING
_C0 = (_SIZE - 2) * _SV * math.log(_SV) + _CONF * math.log(_CONF)

_N = 1024
_BLOCK_V = 2000
_V0 = 68000
_NUM_BLOCKS = _V0 // _BLOCK_V
_SC_V = _SIZE - _V0
_VR = _SC_V // 32
_CH = 40
_NCH = _VR // _CH

_NC = 2
_NS = 16
_NW = _NC * _NS
_ROWS_PER_W = _N // _NW
_VECS_PER_W = _ROWS_PER_W // 16


def _tc_body(x_ref, t_ref, out_ref):
    k = pl.program_id(0)
    x = x_ref[...]
    t = t_ref[...]
    mask = (t != _PADDING_IDX).astype(jnp.float32)

    @pl.when(k == 0)
    def _first():
        csum = jnp.sum(x, axis=0, keepdims=True) - x[0:1, :]
        partial = jnp.sum(csum * mask, axis=(0, 1), keepdims=True)
        cnt = jnp.sum(mask, axis=(0, 1), keepdims=True)
        out_ref[0] = _C0 * cnt - _SV * partial

    @pl.when(k != 0)
    def _rest():
        csum = jnp.sum(x, axis=0, keepdims=True)
        partial = jnp.sum(csum * mask, axis=(0, 1), keepdims=True)
        out_ref[0] = -_SV * partial


def _tc_partials(xt, trow):
    return pl.pallas_call(
        _tc_body,
        grid=(_NUM_BLOCKS,),
        in_specs=[
            pl.BlockSpec((_BLOCK_V, _N), lambda k: (k, 0)),
            pl.BlockSpec((1, _N), lambda k: (0, 0)),
        ],
        out_specs=pl.BlockSpec((1, 1, 1), lambda k: (k, 0, 0)),
        out_shape=jax.ShapeDtypeStruct((_NUM_BLOCKS, 1, 1), jnp.float32),
        compiler_params=pltpu.CompilerParams(
            dimension_semantics=("parallel",),
        ),
    )(xt, trow)


def _sc_body(x_hbm, tgt_hbm, gout_hbm, sout_hbm,
             t_v, tile_v, obuf_v, buf0_v, buf1_v, acc_v,
             semg, sem0, sem1):
    wid = lax.axis_index("s") * _NC + lax.axis_index("c")
    base = wid * _ROWS_PER_W
    pltpu.sync_copy(tgt_hbm.at[pl.ds(base, _ROWS_PER_W)], t_v)

    gcopies = []
    scalars = []
    for c in range(_VECS_PER_W):
        t16 = t_v[pl.ds(c * 16, 16)]
        for l in range(16):
            r = c * 16 + l
            t_r = t16[l]
            trow0 = pl.multiple_of(t_r - jnp.bitwise_and(t_r, 7), 8)
            col0 = pl.multiple_of((base + r) - (base + r) % 128, 128)
            scalars.append(t_r)
            gcopies.append(pltpu.async_copy(
                x_hbm.at[pl.ds(trow0, 8), pl.ds(col0, 128)],
                tile_v.at[r], semg))

    row_base = pl.multiple_of(_V0 + wid * _VR, 8)
    z16 = jnp.zeros((16,), jnp.float32)
    for g in range(64):
        acc_v[pl.ds(g * 16, 16)] = z16
    bufs = [buf0_v, buf1_v]
    sems = [sem0, sem1]

    def start(c, parity):
        r0 = pl.multiple_of(row_base + c * _CH, 8)
        return pltpu.async_copy(x_hbm.at[pl.ds(r0, _CH)], bufs[parity],
                                sems[parity])

    def accumulate(buf_v):
        for p in range(4):
            def row_body(r, accs):
                return tuple(accs[g] + buf_v[r, pl.ds((p * 16 + g) * 16, 16)]
                             for g in range(16))
            accs = lax.fori_loop(0, _CH,
                                 row_body, tuple(z16 for _ in range(16)))
            for g in range(16):
                o = (p * 16 + g) * 16
                acc_v[pl.ds(o, 16)] = acc_v[pl.ds(o, 16)] + accs[g]

    def wait_chunk(c, parity):
        r0 = pl.multiple_of(row_base + c * _CH, 8)
        pltpu.make_async_copy(x_hbm.at[pl.ds(r0, _CH)], bufs[parity],
                              sems[parity]).wait()

    start(0, 0)

    def pair_body(i, carry):
        c = 2 * i
        start(c + 1, 1)
        wait_chunk(c, 0)
        accumulate(buf0_v)
        start(c + 2, 0)
        wait_chunk(c + 1, 1)
        accumulate(buf1_v)
        return carry

    lax.fori_loop(0, (_NCH - 1) // 2, pair_body, 0)
    wait_chunk(_NCH - 1, 0)
    accumulate(buf0_v)
    pltpu.sync_copy(acc_v, sout_hbm.at[wid])

    for cp in gcopies:
        cp.wait()
    for r in range(_ROWS_PER_W):
        t_r = scalars[r]
        s_dyn = jnp.bitwise_and(t_r, 7)
        a = ((base + r) % 128) - ((base + r) % 16)
        obuf_v[r] = tile_v[r, s_dyn, pl.ds(a, 16)]
    pltpu.sync_copy(obuf_v, gout_hbm.at[pl.ds(base, _ROWS_PER_W)])


def _sc_all(xt, t32):
    tgt = t32.reshape(_N)
    mesh = plsc.VectorSubcoreMesh(core_axis_name="c", subcore_axis_name="s")
    f = functools.partial(
        pl.kernel,
        mesh=mesh,
        out_type=(jax.ShapeDtypeStruct((_N, 16), jnp.float32),
                  jax.ShapeDtypeStruct((_NW, _N), jnp.float32)),
        scratch_types=[
            pltpu.VMEM((_ROWS_PER_W,), jnp.int32),
            pltpu.VMEM((_ROWS_PER_W, 8, 128), jnp.float32),
            pltpu.VMEM((_ROWS_PER_W, 16), jnp.float32),
            pltpu.VMEM((_CH, _N), jnp.float32),
            pltpu.VMEM((_CH, _N), jnp.float32),
            pltpu.VMEM((_N,), jnp.float32),
            pltpu.SemaphoreType.DMA,
            pltpu.SemaphoreType.DMA,
            pltpu.SemaphoreType.DMA,
        ],
    )(_sc_body)
    return f(xt, tgt)


def _comb_body(g_ref, scs_ref, t_ref, trow_ref, tc_ref, out_ref):
    t = t_ref[...]
    mask = (t != _PADDING_IDX).astype(jnp.float32)
    g = g_ref[...]
    rows = jax.lax.broadcasted_iota(jnp.int32, g.shape, 0)
    lanes = jax.lax.broadcasted_iota(jnp.int32, g.shape, 1)
    sel = (lanes == rows % 16).astype(jnp.float32)
    o_t = jnp.sum(g * sel, axis=1, keepdims=True)
    corr = (_SV - _CONF) * jnp.sum(o_t * mask, axis=(0, 1), keepdims=True)
    scs = jnp.sum(scs_ref[...], axis=0, keepdims=True)
    maskr = (trow_ref[...] != _PADDING_IDX).astype(jnp.float32)
    sc_part = jnp.sum(scs * maskr, axis=(0, 1), keepdims=True)
    tc_total = jnp.sum(tc_ref[...], axis=(0, 1), keepdims=True)
    out_ref[...] = corr - _SV * sc_part + tc_total


def _combine(gathered, scs, t32, trow, tc):
    return pl.pallas_call(
        _comb_body,
        grid=(1,),
        in_specs=[
            pl.BlockSpec((_N, 16), lambda k: (0, 0)),
            pl.BlockSpec((_NW, _N), lambda k: (0, 0)),
            pl.BlockSpec((_N, 1), lambda k: (0, 0)),
            pl.BlockSpec((1, _N), lambda k: (0, 0)),
            pl.BlockSpec((_NUM_BLOCKS, 1), lambda k: (0, 0)),
        ],
        out_specs=pl.BlockSpec((1, 1), lambda k: (0, 0)),
        out_shape=jax.ShapeDtypeStruct((1, 1), jnp.float32),
    )(gathered, scs, t32, trow, tc)


@jax.jit
def kernel(output, target):
    t32 = target.astype(jnp.int32)
    xt = output.T
    trow = t32.reshape(1, _N)
    gathered, scs = _sc_all(xt, t32)
    tc = _tc_partials(xt, trow)
    comb = _combine(gathered, scs, t32, trow, tc.reshape(_NUM_BLOCKS, 1))
    return comb[0, 0]


# speedup vs baseline: 1.0728x; 1.0243x over previous
import functools
import math

import jax
import jax.numpy as jnp
from jax import lax
from jax.experimental import pallas as pl
from jax.experimental.pallas import tpu as pltpu
from jax.experimental.pallas import tpu_sc as plsc

_SIZE = 100000
_PADDING_IDX = 0
_LABEL_SMOOTHING = 0.1
_SV = _LABEL_SMOOTHING / (_SIZE - 2)
_CONF = 1.0 - _LABEL_SMOOTHING
_C0 = (_SIZE - 2) * _SV * math.log(_SV) + _CONF * math.log(_CONF)

_N = 1024
_BLOCK_V = 2000
_NUM_BLOCKS = _SIZE // _BLOCK_V

_NC = 2
_NS = 16
_NW = _NC * _NS
_ROWS_PER_W = _N // _NW
_VECS_PER_W = _ROWS_PER_W // 16


def _tc_body(x_ref, t_ref, out_ref):
    k = pl.program_id(0)
    x = x_ref[...]
    t = t_ref[...]
    mask = (t != _PADDING_IDX).astype(jnp.float32)

    @pl.when(k == 0)
    def _first():
        csum = jnp.sum(x, axis=0, keepdims=True) - x[0:1, :]
        partial = jnp.sum(csum * mask, axis=(0, 1), keepdims=True)
        cnt = jnp.sum(mask, axis=(0, 1), keepdims=True)
        out_ref[0] = _C0 * cnt - _SV * partial

    @pl.when(k != 0)
    def _rest():
        csum = jnp.sum(x, axis=0, keepdims=True)
        partial = jnp.sum(csum * mask, axis=(0, 1), keepdims=True)
        out_ref[0] = -_SV * partial


def _tc_partials(xt, trow):
    return pl.pallas_call(
        _tc_body,
        grid=(_NUM_BLOCKS,),
        in_specs=[
            pl.BlockSpec((_BLOCK_V, _N), lambda k: (k, 0)),
            pl.BlockSpec((1, _N), lambda k: (0, 0)),
        ],
        out_specs=pl.BlockSpec((1, 1, 1), lambda k: (k, 0, 0)),
        out_shape=jax.ShapeDtypeStruct((_NUM_BLOCKS, 1, 1), jnp.float32),
        compiler_params=pltpu.CompilerParams(
            dimension_semantics=("parallel",),
        ),
    )(xt, trow)


def _sc_gather_body(x_hbm, tgt_hbm, out_hbm, t_v, tile_v, obuf_v, sem):
    wid = lax.axis_index("s") * _NC + lax.axis_index("c")
    base = wid * _ROWS_PER_W
    pltpu.sync_copy(tgt_hbm.at[pl.ds(base, _ROWS_PER_W)], t_v)
    copies = []
    scalars = []
    for c in range(_VECS_PER_W):
        t16 = t_v[pl.ds(c * 16, 16)]
        for l in range(16):
            r = c * 16 + l
            t_r = t16[l]
            trow0 = pl.multiple_of(t_r - jnp.bitwise_and(t_r, 7), 8)
            col0 = pl.multiple_of((base + r) - (base + r) % 128, 128)
            scalars.append(t_r)
            copies.append(pltpu.async_copy(
                x_hbm.at[pl.ds(trow0, 8), pl.ds(col0, 128)],
                tile_v.at[r], sem))
    for cp in copies:
        cp.wait()
    for r in range(_ROWS_PER_W):
        t_r = scalars[r]
        s_dyn = jnp.bitwise_and(t_r, 7)
        a = ((base + r) % 128) - ((base + r) % 16)
        obuf_v[r] = tile_v[r, s_dyn, pl.ds(a, 16)]
    pltpu.sync_copy(obuf_v, out_hbm.at[pl.ds(base, _ROWS_PER_W)])


def _sc_gather(xt, t32):
    tgt = t32.reshape(_N)
    mesh = plsc.VectorSubcoreMesh(core_axis_name="c", subcore_axis_name="s")
    f = functools.partial(
        pl.kernel,
        mesh=mesh,
        out_type=jax.ShapeDtypeStruct((_N, 16), jnp.float32),
        scratch_types=[
            pltpu.VMEM((_ROWS_PER_W,), jnp.int32),
            pltpu.VMEM((_ROWS_PER_W, 8, 128), jnp.float32),
            pltpu.VMEM((_ROWS_PER_W, 16), jnp.float32),
            pltpu.SemaphoreType.DMA,
        ],
    )(_sc_gather_body)
    return f(xt, tgt)


def _comb_body(g_ref, t_ref, out_ref):
    t = t_ref[...]
    mask = (t != _PADDING_IDX).astype(jnp.float32)
    g = g_ref[...]
    rows = jax.lax.broadcasted_iota(jnp.int32, g.shape, 0)
    lanes = jax.lax.broadcasted_iota(jnp.int32, g.shape, 1)
    sel = (lanes == rows % 16).astype(jnp.float32)
    o_t = jnp.sum(g * sel, axis=1, keepdims=True)
    out_ref[...] = (_SV - _CONF) * jnp.sum(
        o_t * mask, axis=(0, 1), keepdims=True)


def _combine(gathered, t32):
    return pl.pallas_call(
        _comb_body,
        grid=(1,),
        in_specs=[
            pl.BlockSpec((_N, 16), lambda k: (0, 0)),
            pl.BlockSpec((_N, 1), lambda k: (0, 0)),
        ],
        out_specs=pl.BlockSpec((1, 1), lambda k: (0, 0)),
        out_shape=jax.ShapeDtypeStruct((1, 1), jnp.float32),
    )(gathered, t32)


@jax.jit
def kernel(output, target):
    t32 = target.astype(jnp.int32)
    xt = output.T
    trow = t32.reshape(1, _N)
    gathered = _sc_gather(xt, t32)
    tc = _tc_partials(xt, trow)
    comb = _combine(gathered, t32)
    return jnp.sum(tc) + comb[0, 0]
